# Initial kernel scaffold; baseline (speedup 1.0000x reference)
import jax
import jax.numpy as jnp
from jax.experimental import pallas as pl


def kernel(z_atom, dist, edge_i, edge_j, M01, M02, idx_ji, idx_ki, emb, W1, b1, W2, b2, Wc, bc, Ws, bs, Wm1, bm1, Wm2, bm2, W01, b01, W02, b02, Wls, bls, Wmo, bmo, Wio, bio, We1, be1, We2, be2):
    raise NotImplementedError("write your pallas kernel here")


# speedup vs baseline: 1.2318x; 1.2318x over previous
import functools

import jax
import jax.numpy as jnp
from jax import lax
from jax.experimental import pallas as pl
from jax.experimental.pallas import tpu as pltpu
from jax.experimental.pallas import tpu_sc as plsc

F = 128
BETA = 8
NND = 10000
ED = 160000
TD = 320000
R_CUT = 5.0

NC = 2
NS = 16
NW = NC * NS

EC = ED // 128
TC_ = TD // 128

_MESH = dict(core_axis_name="c", subcore_axis_name="s")


def _silu(x):
    return x * (1.0 / (1.0 + jnp.exp(-x)))


def _nodes_body(z_ref, emb_ref, w1_ref, b1_ref, w2_ref, b2_ref, x_ref, zz_ref):
    z = z_ref[...]
    oh = (z == lax.broadcasted_iota(jnp.int32, (z.shape[0], 100), 1)).astype(jnp.float32)
    x = jnp.dot(oh, emb_ref[...], preferred_element_type=jnp.float32)
    h = _silu(jnp.dot(x, w1_ref[...], preferred_element_type=jnp.float32) + b1_ref[...])
    zz = jnp.dot(h, w2_ref[...], preferred_element_type=jnp.float32) + b2_ref[...]
    x_ref[...] = x
    zz_ref[...] = zz


def _tc_nodes(z2, emb, w1, b1, w2, b2):
    BN = 1000
    grid = NND // BN
    full = lambda *s: pl.BlockSpec(s, lambda i: (0,) * len(s))
    return pl.pallas_call(
        _nodes_body,
        grid=(grid,),
        in_specs=[
            pl.BlockSpec((BN, 1), lambda i: (i, 0)),
            full(100, F), full(F, F), full(1, F), full(F, F), full(1, F),
        ],
        out_specs=[pl.BlockSpec((BN, F), lambda i: (i, 0))] * 2,
        out_shape=[jax.ShapeDtypeStruct((NND, F), jnp.float32)] * 2,
    )(z2, emb, w1, b1, w2, b2)


def _edges_body(zi_ref, zj_ref, d_ref, wc_ref, bc_ref, ws_ref, bs_ref,
                wm1_ref, bm1_ref, wm2_ref, bm2_ref,
                dz_ref, f01_ref, f02_ref):
    d = d_ref[...]
    t = 2.0 * d / R_CUT - 1.0
    cut = (d - R_CUT) ** 2 * (d < R_CUT).astype(jnp.float32)
    zcat = jnp.concatenate([zi_ref[...], zj_ref[...]], axis=1)
    wc = wc_ref[...]
    bc = bc_ref[...]
    tkm2 = jnp.ones_like(t)
    tkm1 = t
    cq = jnp.zeros((zcat.shape[0], F), jnp.float32)
    for b in range(BETA):
        if b == 0:
            tb = tkm2
        elif b == 1:
            tb = tkm1
        else:
            tb = 2.0 * t * tkm1 - tkm2
            tkm2, tkm1 = tkm1, tb
        qb = tb * cut
        cb = jnp.dot(zcat, wc[b], preferred_element_type=jnp.float32) + bc[b][None, :]
        cq = cq + qb * cb
    sp = jnp.dot(cq, ws_ref[...], preferred_element_type=jnp.float32) + bs_ref[...]
    dz_ref[...] = sp[:, :F]
    f = sp[:, F:]
    hm = _silu(jnp.dot(f, wm1_ref[...], preferred_element_type=jnp.float32) + bm1_ref[...])
    fm = _silu(jnp.dot(hm, wm2_ref[...], preferred_element_type=jnp.float32) + bm2_ref[...])
    f01_ref[...] = fm[:, :F]
    f02_ref[...] = fm[:, F:]


def _tc_edges(zi, zj, dist, wc_r, bc_r, ws, bs, wm1, bm1, wm2, bm2):
    BE = 640
    grid = ED // BE
    full = lambda *s: pl.BlockSpec(s, lambda i: (0,) * len(s))
    row = lambda k: pl.BlockSpec((BE, k), lambda i: (i, 0))
    return pl.pallas_call(
        _edges_body,
        grid=(grid,),
        in_specs=[
            row(F), row(F), pl.BlockSpec((BE, 1), lambda i: (i, 0)),
            full(BETA, 2 * F, F), full(BETA, F),
            full(F, 3 * F), full(1, 3 * F),
            full(2 * F, 2 * F), full(1, 2 * F),
            full(2 * F, 2 * F), full(1, 2 * F),
        ],
        out_specs=[row(F)] * 3,
        out_shape=[jax.ShapeDtypeStruct((ED, F), jnp.float32)] * 3,
    )(zi, zj, dist, wc_r, bc_r, ws, bs, wm1, bm1, wm2, bm2)


def _triplets_body(g1_ref, g2_ref, w01_ref, b01_ref, w02_ref, b02_ref,
                   wla_ref, wlb_ref, bls_ref, st_ref):
    s01 = _silu(jnp.dot(g1_ref[...], w01_ref[...], preferred_element_type=jnp.float32) + b01_ref[...])
    s02 = _silu(jnp.dot(g2_ref[...], w02_ref[...], preferred_element_type=jnp.float32) + b02_ref[...])
    s = (jnp.dot(s01, wla_ref[...], preferred_element_type=jnp.float32)
         + jnp.dot(s02, wlb_ref[...], preferred_element_type=jnp.float32) + bls_ref[...])
    st_ref[...] = _silu(s)


def _tc_triplets(g1, g2, w01, b01, w02, b02, wla, wlb, bls):
    BT = 640
    grid = TD // BT
    full = lambda *s: pl.BlockSpec(s, lambda i: (0,) * len(s))
    row = lambda k: pl.BlockSpec((BT, k), lambda i: (i, 0))
    return pl.pallas_call(
        _triplets_body,
        grid=(grid,),
        in_specs=[row(F), row(F), full(F, F), full(1, F), full(F, F), full(1, F),
                  full(F, F), full(F, F), full(1, F)],
        out_specs=row(F),
        out_shape=jax.ShapeDtypeStruct((TD, F), jnp.float32),
    )(g1, g2, w01, b01, w02, b02, wla, wlb, bls)


def _edges2_body(dz_ref, mo_ref, wmo_ref, bmo_ref, out_ref):
    ba = jnp.dot(mo_ref[...], wmo_ref[...], preferred_element_type=jnp.float32) + bmo_ref[...]
    out_ref[...] = _silu(dz_ref[...] + ba)


def _tc_edges2(dz, mo, wmo, bmo):
    BE = 640
    grid = ED // BE
    full = lambda *s: pl.BlockSpec(s, lambda i: (0,) * len(s))
    row = pl.BlockSpec((BE, F), lambda i: (i, 0))
    return pl.pallas_call(
        _edges2_body,
        grid=(grid,),
        in_specs=[row, row, full(F, F), full(1, F)],
        out_specs=row,
        out_shape=jax.ShapeDtypeStruct((ED, F), jnp.float32),
    )(dz, mo, wmo, bmo)


def _final_body(x_ref, agg_ref, wio_ref, bio_ref, we1_ref, be1_ref,
                we2_ref, be2_ref, e_ref):
    xx = x_ref[...] + jnp.dot(agg_ref[...], wio_ref[...], preferred_element_type=jnp.float32) + bio_ref[...]
    h = _silu(jnp.dot(xx, we1_ref[...], preferred_element_type=jnp.float32) + be1_ref[...])
    e_ref[...] = jnp.dot(h, we2_ref[...], preferred_element_type=jnp.float32) + be2_ref[...]


def _tc_final(x, agg, wio, bio, we1, be1, we2, be2):
    BN = 1000
    grid = NND // BN
    full = lambda *s: pl.BlockSpec(s, lambda i: (0,) * len(s))
    row = pl.BlockSpec((BN, F), lambda i: (i, 0))
    return pl.pallas_call(
        _final_body,
        grid=(grid,),
        in_specs=[row, row, full(F, F), full(1, F), full(F, F), full(1, F),
                  full(F, 1), full(1, 1)],
        out_specs=pl.BlockSpec((BN, 1), lambda i: (i, 0)),
        out_shape=jax.ShapeDtypeStruct((NND, 1), jnp.float32),
    )(x, agg, wio, bio, we1, be1, we2, be2)


def _wid():
    return lax.axis_index("s") * NC + lax.axis_index("c")


def _sc_gather_zz(zz, eij3):
    njm = (EC + NW - 1) // NW

    def body(zz_h, idx_h, zi_h, zj_h, ib, ra, rb, sa, sb):
        w = _wid()

        def step(j, carry):
            c = w + NW * j

            @pl.when(c < EC)
            def _():
                pltpu.sync_copy(idx_h.at[c], ib)
                ca = pltpu.async_copy(zz_h.at[ib.at[0]], ra, sa)
                cb = pltpu.async_copy(zz_h.at[ib.at[1]], rb, sb)
                ca.wait()
                pltpu.sync_copy(ra, zi_h.at[c])
                cb.wait()
                pltpu.sync_copy(rb, zj_h.at[c])
            return carry

        lax.fori_loop(0, njm, step, 0)

    f = pl.kernel(
        body,
        out_type=(jax.ShapeDtypeStruct((EC, 128, F), jnp.float32),) * 2,
        mesh=plsc.VectorSubcoreMesh(**_MESH),
        scratch_types=[
            pltpu.VMEM((2, 128), jnp.int32),
            pltpu.VMEM((128, F), jnp.float32),
            pltpu.VMEM((128, F), jnp.float32),
            pltpu.SemaphoreType.DMA,
            pltpu.SemaphoreType.DMA,
        ],
    )
    return f(zz, eij3)


def _sc_tri_gather(f01, f02, m1r, m2r, kj3):
    njm = (TC_ + NW - 1) // NW

    def body(f01_h, f02_h, m1_h, m2_h, idx_h, g1_h, g2_h,
             ib, abuf, bbuf, mbuf, sa, sb, sm):
        w = _wid()

        def step(j, carry):
            c = w + NW * j

            @pl.when(c < TC_)
            def _():
                pltpu.sync_copy(idx_h.at[c], ib)
                for (ftab, mtab, gout) in ((f01_h, m1_h, g1_h), (f02_h, m2_h, g2_h)):
                    ca = pltpu.async_copy(ftab.at[ib.at[0]], abuf, sa)
                    cb = pltpu.async_copy(ftab.at[ib.at[1]], bbuf, sb)
                    cm = pltpu.async_copy(mtab.at[c], mbuf, sm)
                    ca.wait()
                    cb.wait()
                    cm.wait()

                    def ew(r, cc):
                        for k in range(F // 16):
                            sl = pl.ds(16 * k, 16)
                            abuf[r, sl] = abuf[r, sl] * mbuf[r, sl] * bbuf[r, sl]
                        return cc

                    lax.fori_loop(0, 128, ew, 0)
                    pltpu.sync_copy(abuf, gout.at[c])
            return carry

        lax.fori_loop(0, njm, step, 0)

    f = pl.kernel(
        body,
        out_type=(jax.ShapeDtypeStruct((TC_, 128, F), jnp.float32),) * 2,
        mesh=plsc.VectorSubcoreMesh(**_MESH),
        scratch_types=[
            pltpu.VMEM((2, 128), jnp.int32),
            pltpu.VMEM((128, F), jnp.float32),
            pltpu.VMEM((128, F), jnp.float32),
            pltpu.VMEM((128, F), jnp.float32),
            pltpu.SemaphoreType.DMA,
            pltpu.SemaphoreType.DMA,
            pltpu.SemaphoreType.DMA,
        ],
    )
    return f(f01, f02, m1r, m2r, kj3)


def _sc_scatter(vals3, idx3, nseg, nfeat_sc, zsrc):
    nch = vals3.shape[0]
    cb = 4 if nch % 4 == 0 else 2
    ng = nch // cb
    njm = (ng + NS - 1) // NS
    npass = 64 // nfeat_sc
    zrows = zsrc.shape[0]
    nzcp = nseg // (NS * zrows)
    frows = nseg // NS

    def body2(v_h, i_h, z_h, out_h, acc, ib0, sb0, ib1, sb1, zb, s0, s1):
        sc = lax.axis_index("c")
        sid = lax.axis_index("s")
        pltpu.sync_copy(z_h, zb)

        for p in range(npass):
            fb = sc * 64 + p * nfeat_sc
            for k in range(nzcp):
                pltpu.sync_copy(zb, acc.at[pl.ds(sid * frows + k * zrows, zrows)])
            plsc.subcore_barrier()

            def start(ibuf, sbuf, sem, g):
                @pl.when(g < ng)
                def _():
                    c0 = g * cb
                    pltpu.sync_copy(i_h.at[pl.ds(c0, cb)], ibuf)
                    pltpu.async_copy(v_h.at[pl.ds(c0, cb), :, pl.ds(fb, nfeat_sc)], sbuf, sem)

            def finish(ibuf, sbuf, sem, g):
                @pl.when(g < ng)
                def _():
                    c0 = g * cb
                    pltpu.make_async_copy(
                        v_h.at[pl.ds(c0, cb), :, pl.ds(fb, nfeat_sc)], sbuf, sem).wait()
                    for r in range(cb):
                        pltpu.sync_copy(sbuf.at[r], acc.at[ibuf.at[r]], add=True)

            start(ib0, sb0, s0, sid)

            def step(t, carry):
                ga = sid + NS * (2 * t)
                gb = sid + NS * (2 * t + 1)
                start(ib1, sb1, s1, gb)
                finish(ib0, sb0, s0, ga)
                start(ib0, sb0, s0, sid + NS * (2 * t + 2))
                finish(ib1, sb1, s1, gb)
                return carry

            lax.fori_loop(0, (njm + 1) // 2, step, 0)
            plsc.subcore_barrier()
            pltpu.sync_copy(
                acc.at[pl.ds(sid * frows, frows)],
                out_h.at[pl.ds(sid * frows, frows), pl.ds(fb, nfeat_sc)])
            plsc.subcore_barrier()

    f = pl.kernel(
        body2,
        out_type=jax.ShapeDtypeStruct((nseg, F), jnp.float32),
        mesh=plsc.VectorSubcoreMesh(**_MESH),
        compiler_params=pltpu.CompilerParams(use_tc_tiling_on_sc=False),
        scratch_types=[
            pltpu.VMEM_SHARED((nseg, nfeat_sc), jnp.float32),
            pltpu.VMEM((cb, 128), jnp.int32),
            pltpu.VMEM((cb, 128, nfeat_sc), jnp.float32),
            pltpu.VMEM((cb, 128), jnp.int32),
            pltpu.VMEM((cb, 128, nfeat_sc), jnp.float32),
            pltpu.VMEM((zrows, nfeat_sc), jnp.float32),
            pltpu.SemaphoreType.DMA,
            pltpu.SemaphoreType.DMA,
        ],
    )
    return f(vals3, idx3, zsrc)


def kernel(z_atom, dist, edge_i, edge_j, M01, M02, idx_ji, idx_ki, emb,
           W1, b1, W2, b2, Wc, bc, Ws, bs, Wm1, bm1, Wm2, bm2,
           W01, b01, W02, b02, Wls, bls, Wmo, bmo, Wio, bio, We1, be1, We2, be2):
    f32 = jnp.float32
    r1 = lambda b: b.reshape(1, -1).astype(f32)

    z2 = z_atom.reshape(NND, 1).astype(jnp.int32)
    eij3 = jnp.stack([edge_i.reshape(EC, 128), edge_j.reshape(EC, 128)], axis=1).astype(jnp.int32)
    kj3 = jnp.stack([idx_ki.reshape(TC_, 128), idx_ji.reshape(TC_, 128)], axis=1).astype(jnp.int32)
    ji3 = idx_ji.reshape(TC_, 128).astype(jnp.int32)
    ei3 = edge_i.reshape(EC, 128).astype(jnp.int32)
    m1r = M01.reshape(TC_, 128, F)
    m2r = M02.reshape(TC_, 128, F)
    wc_r = jnp.transpose(Wc.reshape(2 * F, F, BETA), (2, 0, 1))
    bc_r = bc.reshape(F, BETA).T
    wla = Wls[:F]
    wlb = Wls[F:]

    x, zz = _tc_nodes(z2, emb, W1, r1(b1), W2, r1(b2))

    zi3, zj3 = _sc_gather_zz(zz, eij3)
    zi = zi3.reshape(ED, F)
    zj = zj3.reshape(ED, F)

    dz, f01, f02 = _tc_edges(zi, zj, dist, wc_r, bc_r, Ws, r1(bs),
                             Wm1, r1(bm1), Wm2, r1(bm2))

    g13, g23 = _sc_tri_gather(f01, f02, m1r, m2r, kj3)
    g1 = g13.reshape(TD, F)
    g2 = g23.reshape(TD, F)

    st = _tc_triplets(g1, g2, W01, r1(b01), W02, r1(b02), wla, wlb, r1(bls))

    zsrc_mo = jnp.zeros((1250, 8), f32)
    mo = _sc_scatter(st.reshape(TC_, 128, F), ji3, ED, 8, zsrc_mo)

    out = _tc_edges2(dz, mo, Wmo, r1(bmo))

    zsrc_agg = jnp.zeros((625, 64), f32)
    agg = _sc_scatter(out.reshape(EC, 128, F), ei3, NND, 64, zsrc_agg)

    e = _tc_final(x, agg, Wio, r1(bio), We1, r1(be1), We2.reshape(F, 1), r1(be2))
    return e


# speedup vs baseline: 1.3820x; 1.1219x over previous
import functools

import jax
import jax.numpy as jnp
from jax import lax
from jax.experimental import pallas as pl
from jax.experimental.pallas import tpu as pltpu
from jax.experimental.pallas import tpu_sc as plsc

F = 128
BETA = 8
NND = 10000
ED = 160000
TD = 320000
R_CUT = 5.0

NC = 2
NS = 16
NW = NC * NS

EC = ED // 128
TC_ = TD // 128

_MESH = dict(core_axis_name="c", subcore_axis_name="s")


def _silu(x):
    return x * (1.0 / (1.0 + jnp.exp(-x)))


def _nodes_body(z_ref, emb_ref, w1_ref, b1_ref, w2_ref, b2_ref, x_ref, zz_ref):
    z = z_ref[...]
    oh = (z == lax.broadcasted_iota(jnp.int32, (z.shape[0], 100), 1)).astype(jnp.float32)
    x = jnp.dot(oh, emb_ref[...], preferred_element_type=jnp.float32)
    h = _silu(jnp.dot(x, w1_ref[...], preferred_element_type=jnp.float32) + b1_ref[...])
    zz = jnp.dot(h, w2_ref[...], preferred_element_type=jnp.float32) + b2_ref[...]
    x_ref[...] = x
    zz_ref[...] = zz


def _tc_nodes(z2, emb, w1, b1, w2, b2):
    BN = 1000
    grid = NND // BN
    full = lambda *s: pl.BlockSpec(s, lambda i: (0,) * len(s))
    return pl.pallas_call(
        _nodes_body,
        grid=(grid,),
        in_specs=[
            pl.BlockSpec((BN, 1), lambda i: (i, 0)),
            full(100, F), full(F, F), full(1, F), full(F, F), full(1, F),
        ],
        out_specs=[pl.BlockSpec((BN, F), lambda i: (i, 0))] * 2,
        out_shape=[jax.ShapeDtypeStruct((NND, F), jnp.float32)] * 2,
    )(z2, emb, w1, b1, w2, b2)


def _edges_body(zi_ref, zj_ref, d_ref, wc_ref, bc_ref, ws_ref, bs_ref,
                wm1_ref, bm1_ref, wm2_ref, bm2_ref,
                dz_ref, f01_ref, f02_ref):
    d = d_ref[...]
    t = 2.0 * d / R_CUT - 1.0
    cut = (d - R_CUT) ** 2 * (d < R_CUT).astype(jnp.float32)
    zcat = jnp.concatenate([zi_ref[...], zj_ref[...]], axis=1)
    wc = wc_ref[...]
    bc = bc_ref[...]
    tkm2 = jnp.ones_like(t)
    tkm1 = t
    cq = jnp.zeros((zcat.shape[0], F), jnp.float32)
    for b in range(BETA):
        if b == 0:
            tb = tkm2
        elif b == 1:
            tb = tkm1
        else:
            tb = 2.0 * t * tkm1 - tkm2
            tkm2, tkm1 = tkm1, tb
        qb = tb * cut
        cb = jnp.dot(zcat, wc[b], preferred_element_type=jnp.float32) + bc[b][None, :]
        cq = cq + qb * cb
    sp = jnp.dot(cq, ws_ref[...], preferred_element_type=jnp.float32) + bs_ref[...]
    dz_ref[...] = sp[:, :F]
    f = sp[:, F:]
    hm = _silu(jnp.dot(f, wm1_ref[...], preferred_element_type=jnp.float32) + bm1_ref[...])
    fm = _silu(jnp.dot(hm, wm2_ref[...], preferred_element_type=jnp.float32) + bm2_ref[...])
    f01_ref[...] = fm[:, :F]
    f02_ref[...] = fm[:, F:]


def _tc_edges(zi, zj, dist, wc_r, bc_r, ws, bs, wm1, bm1, wm2, bm2):
    BE = 640
    grid = ED // BE
    full = lambda *s: pl.BlockSpec(s, lambda i: (0,) * len(s))
    row = lambda k: pl.BlockSpec((BE, k), lambda i: (i, 0))
    return pl.pallas_call(
        _edges_body,
        grid=(grid,),
        in_specs=[
            row(F), row(F), pl.BlockSpec((BE, 1), lambda i: (i, 0)),
            full(BETA, 2 * F, F), full(BETA, F),
            full(F, 3 * F), full(1, 3 * F),
            full(2 * F, 2 * F), full(1, 2 * F),
            full(2 * F, 2 * F), full(1, 2 * F),
        ],
        out_specs=[row(F)] * 3,
        out_shape=[jax.ShapeDtypeStruct((ED, F), jnp.float32)] * 3,
    )(zi, zj, dist, wc_r, bc_r, ws, bs, wm1, bm1, wm2, bm2)


def _triplets_body(g1_ref, g2_ref, w01_ref, b01_ref, w02_ref, b02_ref,
                   wla_ref, wlb_ref, bls_ref, st_ref):
    s01 = _silu(jnp.dot(g1_ref[...], w01_ref[...], preferred_element_type=jnp.float32) + b01_ref[...])
    s02 = _silu(jnp.dot(g2_ref[...], w02_ref[...], preferred_element_type=jnp.float32) + b02_ref[...])
    s = (jnp.dot(s01, wla_ref[...], preferred_element_type=jnp.float32)
         + jnp.dot(s02, wlb_ref[...], preferred_element_type=jnp.float32) + bls_ref[...])
    st_ref[...] = _silu(s)


def _tc_triplets(g1, g2, w01, b01, w02, b02, wla, wlb, bls):
    BT = 640
    grid = TD // BT
    full = lambda *s: pl.BlockSpec(s, lambda i: (0,) * len(s))
    row = lambda k: pl.BlockSpec((BT, k), lambda i: (i, 0))
    return pl.pallas_call(
        _triplets_body,
        grid=(grid,),
        in_specs=[row(F), row(F), full(F, F), full(1, F), full(F, F), full(1, F),
                  full(F, F), full(F, F), full(1, F)],
        out_specs=row(F),
        out_shape=jax.ShapeDtypeStruct((TD, F), jnp.float32),
    )(g1, g2, w01, b01, w02, b02, wla, wlb, bls)


def _edges2_body(dz_ref, mo_ref, wmo_ref, bmo_ref, out_ref):
    ba = jnp.dot(mo_ref[...], wmo_ref[...], preferred_element_type=jnp.float32) + bmo_ref[...]
    out_ref[...] = _silu(dz_ref[...] + ba)


def _tc_edges2(dz, mo, wmo, bmo):
    BE = 640
    grid = ED // BE
    full = lambda *s: pl.BlockSpec(s, lambda i: (0,) * len(s))
    row = pl.BlockSpec((BE, F), lambda i: (i, 0))
    return pl.pallas_call(
        _edges2_body,
        grid=(grid,),
        in_specs=[row, row, full(F, F), full(1, F)],
        out_specs=row,
        out_shape=jax.ShapeDtypeStruct((ED, F), jnp.float32),
    )(dz, mo, wmo, bmo)


def _final_body(x_ref, agg_ref, wio_ref, bio_ref, we1_ref, be1_ref,
                we2_ref, be2_ref, e_ref):
    xx = x_ref[...] + jnp.dot(agg_ref[...], wio_ref[...], preferred_element_type=jnp.float32) + bio_ref[...]
    h = _silu(jnp.dot(xx, we1_ref[...], preferred_element_type=jnp.float32) + be1_ref[...])
    e_ref[...] = jnp.dot(h, we2_ref[...], preferred_element_type=jnp.float32) + be2_ref[...]


def _tc_final(x, agg, wio, bio, we1, be1, we2, be2):
    BN = 1000
    grid = NND // BN
    full = lambda *s: pl.BlockSpec(s, lambda i: (0,) * len(s))
    row = pl.BlockSpec((BN, F), lambda i: (i, 0))
    return pl.pallas_call(
        _final_body,
        grid=(grid,),
        in_specs=[row, row, full(F, F), full(1, F), full(F, F), full(1, F),
                  full(F, 1), full(1, 1)],
        out_specs=pl.BlockSpec((BN, 1), lambda i: (i, 0)),
        out_shape=jax.ShapeDtypeStruct((NND, 1), jnp.float32),
    )(x, agg, wio, bio, we1, be1, we2, be2)


def _wid():
    return lax.axis_index("s") * NC + lax.axis_index("c")


def _sc_gather_zz(zz, eij4):
    njm = eij4.shape[0]

    def body(zz_h, idx_h, zi_h, zj_h, iball, ra0, rb0, ra1, rb1,
             sa0, sb0, sa1, sb1, wa0, wb0, wa1, wb1):
        w = _wid()
        pltpu.sync_copy(idx_h.at[:, w], iball)

        def guard(j):
            c = w + NW * j
            return jnp.logical_and(c >= 0, c < EC)

        def start(ra, rb, sa, sb, j):
            @pl.when(guard(j))
            def _():
                pltpu.async_copy(zz_h.at[iball.at[j, 0]], ra, sa)
                pltpu.async_copy(zz_h.at[iball.at[j, 1]], rb, sb)

        def wait_g(ra, rb, sa, sb, j):
            @pl.when(guard(j))
            def _():
                pltpu.make_async_copy(zz_h.at[iball.at[j, 0]], ra, sa).wait()
                pltpu.make_async_copy(zz_h.at[iball.at[j, 1]], rb, sb).wait()

        def put(ra, rb, wa, wb, j):
            c = w + NW * j

            @pl.when(guard(j))
            def _():
                pltpu.async_copy(ra, zi_h.at[c], wa)
                pltpu.async_copy(rb, zj_h.at[c], wb)

        def drain_put(ra, rb, wa, wb, j):
            c = w + NW * j

            @pl.when(guard(j))
            def _():
                pltpu.make_async_copy(ra, zi_h.at[c], wa).wait()
                pltpu.make_async_copy(rb, zj_h.at[c], wb).wait()

        start(ra0, rb0, sa0, sb0, 0)

        def step(t, carry):
            j0 = 2 * t
            j1 = 2 * t + 1
            wait_g(ra0, rb0, sa0, sb0, j0)
            drain_put(ra1, rb1, wa1, wb1, j0 - 1)
            start(ra1, rb1, sa1, sb1, j1)
            put(ra0, rb0, wa0, wb0, j0)
            wait_g(ra1, rb1, sa1, sb1, j1)
            drain_put(ra0, rb0, wa0, wb0, j0)
            start(ra0, rb0, sa0, sb0, j0 + 2)
            put(ra1, rb1, wa1, wb1, j1)
            return carry

        lax.fori_loop(0, (njm + 1) // 2, step, 0)
        drain_put(ra1, rb1, wa1, wb1, njm - 1)

    f = pl.kernel(
        body,
        out_type=(jax.ShapeDtypeStruct((EC, 128, F), jnp.float32),) * 2,
        mesh=plsc.VectorSubcoreMesh(**_MESH),
        scratch_types=[
            pltpu.VMEM((njm, 2, 128), jnp.int32),
            pltpu.VMEM((128, F), jnp.float32),
            pltpu.VMEM((128, F), jnp.float32),
            pltpu.VMEM((128, F), jnp.float32),
            pltpu.VMEM((128, F), jnp.float32),
        ] + [pltpu.SemaphoreType.DMA] * 8,
    )
    return f(zz, eij4)


def _sc_tri_gather(f01, f02, m1r, m2r, kj4):
    njm = kj4.shape[0]

    def body(f01_h, f02_h, m1_h, m2_h, idx_h, g1_h, g2_h,
             iball, a0, b0, m0, a1, b1, m1,
             sa0, sb0, sm0, sa1, sb1, sm1, wr0, wr1):
        w = _wid()
        pltpu.sync_copy(idx_h.at[:, w], iball)

        def guard(j):
            c = w + NW * j
            return jnp.logical_and(c >= 0, c < TC_)

        def start(tab, abuf, bbuf, mbuf, sa, sb, sm, j):
            c = w + NW * j
            ftab = f01_h if tab == 0 else f02_h
            mtab = m1_h if tab == 0 else m2_h

            @pl.when(guard(j))
            def _():
                pltpu.async_copy(ftab.at[iball.at[j, 0]], abuf, sa)
                pltpu.async_copy(ftab.at[iball.at[j, 1]], bbuf, sb)
                pltpu.async_copy(mtab.at[c], mbuf, sm)

        def wait_g(tab, abuf, bbuf, mbuf, sa, sb, sm, j):
            c = w + NW * j
            ftab = f01_h if tab == 0 else f02_h
            mtab = m1_h if tab == 0 else m2_h

            @pl.when(guard(j))
            def _():
                pltpu.make_async_copy(ftab.at[iball.at[j, 0]], abuf, sa).wait()
                pltpu.make_async_copy(ftab.at[iball.at[j, 1]], bbuf, sb).wait()
                pltpu.make_async_copy(mtab.at[c], mbuf, sm).wait()

        def compute(abuf, bbuf, mbuf, j):
            @pl.when(guard(j))
            def _():
                def ew(r, cc):
                    for k in range(F // 16):
                        sl = pl.ds(16 * k, 16)
                        abuf[r, sl] = abuf[r, sl] * mbuf[r, sl] * bbuf[r, sl]
                    return cc

                lax.fori_loop(0, 128, ew, 0)

        def put(tab, abuf, wsem, j):
            c = w + NW * j
            gout = g1_h if tab == 0 else g2_h

            @pl.when(guard(j))
            def _():
                pltpu.async_copy(abuf, gout.at[c], wsem)

        def drain_put(tab, abuf, wsem, j):
            c = w + NW * j
            gout = g1_h if tab == 0 else g2_h

            @pl.when(guard(j))
            def _():
                pltpu.make_async_copy(abuf, gout.at[c], wsem).wait()

        start(0, a0, b0, m0, sa0, sb0, sm0, 0)

        def step(t, carry):
            wait_g(0, a0, b0, m0, sa0, sb0, sm0, t)
            drain_put(1, a1, wr1, t - 1)
            start(1, a1, b1, m1, sa1, sb1, sm1, t)
            compute(a0, b0, m0, t)
            put(0, a0, wr0, t)
            wait_g(1, a1, b1, m1, sa1, sb1, sm1, t)
            compute(a1, b1, m1, t)
            put(1, a1, wr1, t)
            drain_put(0, a0, wr0, t)
            start(0, a0, b0, m0, sa0, sb0, sm0, t + 1)
            return carry

        lax.fori_loop(0, njm, step, 0)
        drain_put(1, a1, wr1, njm - 1)

    f = pl.kernel(
        body,
        out_type=(jax.ShapeDtypeStruct((TC_, 128, F), jnp.float32),) * 2,
        mesh=plsc.VectorSubcoreMesh(**_MESH),
        scratch_types=[
            pltpu.VMEM((njm, 2, 128), jnp.int32),
            pltpu.VMEM((128, F), jnp.float32),
            pltpu.VMEM((128, F), jnp.float32),
            pltpu.VMEM((128, F), jnp.float32),
            pltpu.VMEM((128, F), jnp.float32),
            pltpu.VMEM((128, F), jnp.float32),
            pltpu.VMEM((128, F), jnp.float32),
        ] + [pltpu.SemaphoreType.DMA] * 8,
    )
    return f(f01, f02, m1r, m2r, kj4)


def _sc_scatter(vals3, idx4, nseg, nfeat_sc, zsrc, ng):
    njm, _, cb, _ = idx4.shape
    npass = 64 // nfeat_sc
    zrows = zsrc.shape[0]
    nzcp = nseg // (NS * zrows)
    frows = nseg // NS

    def body(v_h, i_h, z_h, out_h, acc, iball, sb0, sb1, zb, sl0, sl1, sa0, sa1):
        sc = lax.axis_index("c")
        sid = lax.axis_index("s")
        pltpu.sync_copy(z_h, zb)
        pltpu.sync_copy(i_h.at[:, sid], iball)

        for p in range(npass):
            fb = sc * 64 + p * nfeat_sc

            def guard(j):
                g = j * NS + sid
                return jnp.logical_and(j >= 0, g < ng)

            def vslice(j):
                g = j * NS + sid
                return v_h.at[pl.ds(g * cb, cb), :, pl.ds(fb, nfeat_sc)]

            def start(sbuf, sem, j):
                @pl.when(guard(j))
                def _():
                    pltpu.async_copy(vslice(j), sbuf, sem)

            def wait_load(sbuf, sem, j):
                @pl.when(guard(j))
                def _():
                    pltpu.make_async_copy(vslice(j), sbuf, sem).wait()

            def adds(sbuf, sem, j):
                @pl.when(guard(j))
                def _():
                    for r in range(cb):
                        pltpu.async_copy(sbuf.at[r], acc.at[iball.at[j, r]], sem, add=True)

            def drain_adds(sbuf, sem, j):
                @pl.when(guard(j))
                def _():
                    for r in range(cb):
                        pltpu.make_async_copy(sbuf.at[r], acc.at[iball.at[j, r]], sem).wait()

            for k in range(nzcp):
                pltpu.sync_copy(zb, acc.at[pl.ds(sid * frows + k * zrows, zrows)])
            plsc.subcore_barrier()

            start(sb0, sl0, 0)

            def step(t, carry):
                j0 = 2 * t
                j1 = 2 * t + 1
                wait_load(sb0, sl0, j0)
                drain_adds(sb1, sa1, j0 - 1)
                start(sb1, sl1, j1)
                adds(sb0, sa0, j0)
                wait_load(sb1, sl1, j1)
                drain_adds(sb0, sa0, j0)
                start(sb0, sl0, j0 + 2)
                adds(sb1, sa1, j1)
                return carry

            lax.fori_loop(0, (njm + 1) // 2, step, 0)
            drain_adds(sb1, sa1, njm - 1)
            plsc.subcore_barrier()
            pltpu.sync_copy(
                acc.at[pl.ds(sid * frows, frows)],
                out_h.at[pl.ds(sid * frows, frows), pl.ds(fb, nfeat_sc)])
            plsc.subcore_barrier()

    f = pl.kernel(
        body,
        out_type=jax.ShapeDtypeStruct((nseg, F), jnp.float32),
        mesh=plsc.VectorSubcoreMesh(**_MESH),
        compiler_params=pltpu.CompilerParams(use_tc_tiling_on_sc=False),
        scratch_types=[
            pltpu.VMEM_SHARED((nseg, nfeat_sc), jnp.float32),
            pltpu.VMEM((njm, cb, 128), jnp.int32),
            pltpu.VMEM((cb, 128, nfeat_sc), jnp.float32),
            pltpu.VMEM((cb, 128, nfeat_sc), jnp.float32),
            pltpu.VMEM((zrows, nfeat_sc), jnp.float32),
            pltpu.SemaphoreType.DMA,
            pltpu.SemaphoreType.DMA,
            pltpu.SemaphoreType.DMA,
            pltpu.SemaphoreType.DMA,
        ],
    )
    return f(vals3, idx4, zsrc)


def kernel(z_atom, dist, edge_i, edge_j, M01, M02, idx_ji, idx_ki, emb,
           W1, b1, W2, b2, Wc, bc, Ws, bs, Wm1, bm1, Wm2, bm2,
           W01, b01, W02, b02, Wls, bls, Wmo, bmo, Wio, bio, We1, be1, We2, be2):
    f32 = jnp.float32
    r1 = lambda b: b.reshape(1, -1).astype(f32)

    z2 = z_atom.reshape(NND, 1).astype(jnp.int32)
    eij3 = jnp.stack([edge_i.reshape(EC, 128), edge_j.reshape(EC, 128)], axis=1).astype(jnp.int32)
    eij4 = jnp.pad(eij3, ((0, 40 * NW - EC), (0, 0), (0, 0))).reshape(40, NW, 2, 128)
    kj3 = jnp.stack([idx_ki.reshape(TC_, 128), idx_ji.reshape(TC_, 128)], axis=1).astype(jnp.int32)
    kj4 = jnp.pad(kj3, ((0, 79 * NW - TC_), (0, 0), (0, 0))).reshape(79, NW, 2, 128)
    ji4 = jnp.pad(idx_ji.reshape(TC_, 128).astype(jnp.int32),
                  ((0, 16 * NS * 10 - TC_), (0, 0))).reshape(16, NS, 10, 128)
    ei4 = jnp.pad(edge_i.reshape(EC, 128).astype(jnp.int32),
                  ((0, 40 * NS * 2 - EC), (0, 0))).reshape(40, NS, 2, 128)
    m1r = M01.reshape(TC_, 128, F)
    m2r = M02.reshape(TC_, 128, F)
    wc_r = jnp.transpose(Wc.reshape(2 * F, F, BETA), (2, 0, 1))
    bc_r = bc.reshape(F, BETA).T
    wla = Wls[:F]
    wlb = Wls[F:]

    x, zz = _tc_nodes(z2, emb, W1, r1(b1), W2, r1(b2))

    zi3, zj3 = _sc_gather_zz(zz, eij4)
    zi = zi3.reshape(ED, F)
    zj = zj3.reshape(ED, F)

    dz, f01, f02 = _tc_edges(zi, zj, dist, wc_r, bc_r, Ws, r1(bs),
                             Wm1, r1(bm1), Wm2, r1(bm2))

    g13, g23 = _sc_tri_gather(f01, f02, m1r, m2r, kj4)
    g1 = g13.reshape(TD, F)
    g2 = g23.reshape(TD, F)

    st = _tc_triplets(g1, g2, W01, r1(b01), W02, r1(b02), wla, wlb, r1(bls))

    zsrc_mo = jnp.zeros((1250, 8), f32)
    mo = _sc_scatter(st.reshape(TC_, 128, F), ji4, ED, 8, zsrc_mo, 250)

    out = _tc_edges2(dz, mo, Wmo, r1(bmo))

    zsrc_agg = jnp.zeros((625, 64), f32)
    agg = _sc_scatter(out.reshape(EC, 128, F), ei4, NND, 64, zsrc_agg, 625)

    e = _tc_final(x, agg, Wio, r1(bio), We1, r1(be1), We2.reshape(F, 1), r1(be2))
    return e
